# baseline (speedup 1.0000x reference)
import functools

import jax
import jax.numpy as jnp
from jax import lax
from jax.experimental import pallas as pl
from jax.experimental.pallas import tpu as pltpu
from jax.experimental.pallas import tpu_sc as plsc

EMBED = 256
HEADS = 8
LEVELS = 4
POINTS = 4
DPH = EMBED // HEADS
SHAPES = [[92, 160], [46, 80], [23, 40], [12, 20]]
NV = sum(h * w for h, w in SHAPES)
BS = 4
NQ = 900
NW = 32
ROWS_PER_Q = LEVELS * POINTS * 4
Q_PER_GROUP = 2
GROUP_ROWS = ROWS_PER_Q * Q_PER_GROUP
GROUPS = NQ // Q_PER_GROUP
CHUNK = 40
GROUPS_PAD = 480
N_CHUNKS = GROUPS_PAD // CHUNK
NQ_PAD = GROUPS_PAD * Q_PER_GROUP
NBUF = 4


def _matmul_bias_kernel(x_ref, w_ref, b_ref, o_ref):
    o_ref[...] = (
        jnp.dot(x_ref[...], w_ref[...], preferred_element_type=jnp.float32)
        + b_ref[...]
    )


def _matmul_bias(x, w, b, block_m):
    m, k = x.shape
    n = w.shape[1]
    assert m % block_m == 0
    return pl.pallas_call(
        _matmul_bias_kernel,
        grid=(m // block_m,),
        in_specs=[
            pl.BlockSpec((block_m, k), lambda i: (i, 0)),
            pl.BlockSpec((k, n), lambda i: (0, 0)),
            pl.BlockSpec((1, n), lambda i: (0, 0)),
        ],
        out_specs=pl.BlockSpec((block_m, n), lambda i: (i, 0)),
        out_shape=jax.ShapeDtypeStruct((m, n), jnp.float32),
    )(x, w, b.reshape(1, n))


def _sc_gather_weighted_sum(table, idx, wts):
    mesh = plsc.VectorSubcoreMesh(core_axis_name="c", subcore_axis_name="s")

    @functools.partial(
        pl.kernel,
        out_type=jax.ShapeDtypeStruct((NW, NQ, DPH), jnp.float32),
        mesh=mesh,
        scratch_types=[
            pltpu.VMEM((2, CHUNK, GROUP_ROWS), jnp.int32),
            pltpu.VMEM((2, CHUNK, GROUP_ROWS), jnp.float32),
            pltpu.VMEM((NBUF, GROUP_ROWS, DPH), jnp.float32),
            pltpu.VMEM((NQ_PAD, DPH), jnp.float32),
            [pltpu.SemaphoreType.DMA] * NBUF,
            [pltpu.SemaphoreType.DMA] * 2,
        ],
        compiler_params=pltpu.CompilerParams(use_tc_tiling_on_sc=False),
    )
    def sc_kernel(table_hbm, idx_hbm, w_hbm, out_hbm,
                  idx_v, w_v, ring, out_v, gsems, ssems):
        wid = lax.axis_index("s") * 2 + lax.axis_index("c")

        def stage_issue(c, par):
            pltpu.async_copy(
                idx_hbm.at[wid, pl.ds(c * CHUNK, CHUNK)], idx_v.at[par],
                ssems[par])
            pltpu.async_copy(
                w_hbm.at[wid, pl.ds(c * CHUNK, CHUNK)], w_v.at[par],
                ssems[par])

        def stage_wait(c, par):
            pltpu.make_async_copy(
                idx_hbm.at[wid, pl.ds(c * CHUNK, CHUNK)], idx_v.at[par],
                ssems[par]).wait()
            pltpu.make_async_copy(
                w_hbm.at[wid, pl.ds(c * CHUNK, CHUNK)], w_v.at[par],
                ssems[par]).wait()

        def gather_issue(ib, g, slot):
            pltpu.async_copy(table_hbm.at[ib.at[g]], ring.at[slot], gsems[slot])

        def gather_wait(ib, g, slot):
            pltpu.make_async_copy(
                table_hbm.at[ib.at[g]], ring.at[slot], gsems[slot]).wait()

        def compute_group(wb, g_local, g_abs, slot):
            for sub in range(Q_PER_GROUP):
                acc0 = jnp.zeros((16,), jnp.float32)
                acc1 = jnp.zeros((16,), jnp.float32)
                for jc in range(ROWS_PER_Q // 16):
                    base = sub * ROWS_PER_Q + jc * 16
                    w16 = wb[g_local, pl.ds(base, 16)]
                    for k in range(16):
                        acc0 = acc0 + w16[k] * ring[slot, base + k, pl.ds(0, 16)]
                        acc1 = acc1 + w16[k] * ring[slot, base + k, pl.ds(16, 16)]
                q_local = g_abs * Q_PER_GROUP + sub
                out_v[q_local, pl.ds(0, 16)] = acc0
                out_v[q_local, pl.ds(16, 16)] = acc1

        def run_chunk(c, par):
            stage_wait(c, par)
            ib = idx_v.at[par]
            wb = w_v.at[par]
            @pl.when(c + 1 < N_CHUNKS)
            def _():
                stage_issue(c + 1, 1 - par)
            for b in range(NBUF - 1):
                gather_issue(ib, b, b)

            def ring_body(i, _):
                for b in range(NBUF):
                    g = i * NBUF + b
                    nxt = g + NBUF - 1
                    @pl.when(nxt < CHUNK)
                    def _():
                        gather_issue(ib, nxt, (b + NBUF - 1) % NBUF)
                    gather_wait(ib, g, b)
                    compute_group(wb, g, c * CHUNK + g, b)
                return 0

            lax.fori_loop(0, CHUNK // NBUF, ring_body, 0)

        stage_issue(0, 0)

        def chunk_pair(c2, _):
            run_chunk(2 * c2, 0)
            run_chunk(2 * c2 + 1, 1)
            return 0

        lax.fori_loop(0, N_CHUNKS // 2, chunk_pair, 0)
        pltpu.sync_copy(out_v.at[pl.ds(0, NQ)], out_hbm.at[wid])

    return sc_kernel(table, idx, wts)


def _build_indices_weights(reference_points, off, aw):
    shapes = jnp.array(SHAPES, dtype=jnp.float32)
    wh = jnp.stack([shapes[:, 1], shapes[:, 0]], axis=-1)
    loc = reference_points[:, :, None, :, None, :] + off / wh[None, None, None, :, None, :]
    x = loc[..., 0] * wh[None, None, None, :, None, 0] - 0.5
    y = loc[..., 1] * wh[None, None, None, :, None, 1] - 0.5
    x0 = jnp.floor(x)
    y0 = jnp.floor(y)
    fx = x - x0
    fy = y - y0
    Wl = wh[None, None, None, :, None, 0]
    Hl = wh[None, None, None, :, None, 1]
    starts = []
    s = 0
    for (H_, W_) in SHAPES:
        starts.append(s)
        s += H_ * W_
    lvl_start = jnp.array(starts, dtype=jnp.float32)[None, None, None, :, None]
    lvl_w = wh[None, None, None, :, None, 0]

    idx_c = []
    wts_c = []
    for (dy, dx, wexpr) in (
            (0.0, 0.0, lambda: (1 - fx) * (1 - fy)),
            (1.0, 0.0, lambda: (1 - fx) * fy),
            (0.0, 1.0, lambda: fx * (1 - fy)),
            (1.0, 1.0, lambda: fx * fy)):
        ix = x0 + dx
        iy = y0 + dy
        valid = ((ix >= 0) & (ix <= Wl - 1) & (iy >= 0) & (iy <= Hl - 1))
        ixc = jnp.clip(ix, 0, Wl - 1)
        iyc = jnp.clip(iy, 0, Hl - 1)
        flat = lvl_start + iyc * lvl_w + ixc
        idx_c.append(flat)
        wts_c.append(wexpr() * valid.astype(jnp.float32))
    flat4 = jnp.stack(idx_c, axis=-1)
    w4 = jnp.stack(wts_c, axis=-1) * aw[..., None]
    b_ix = jnp.arange(BS, dtype=jnp.float32)[:, None, None, None, None, None]
    h_ix = jnp.arange(HEADS, dtype=jnp.float32)[None, None, :, None, None, None]
    rows = (b_ix * NV + flat4) * HEADS + h_ix
    rows = rows.astype(jnp.int32)
    rows = rows.reshape(BS, NQ, HEADS, ROWS_PER_Q).transpose(0, 2, 1, 3)
    w4 = w4.reshape(BS, NQ, HEADS, ROWS_PER_Q).transpose(0, 2, 1, 3)
    rows = rows.reshape(NW, GROUPS, GROUP_ROWS)
    w4 = w4.reshape(NW, GROUPS, GROUP_ROWS)
    pad = ((0, 0), (0, GROUPS_PAD - GROUPS), (0, 0))
    return jnp.pad(rows, pad), jnp.pad(w4, pad)


def kernel(query, value, reference_points, spatial_shapes, W_value, b_value,
           W_off, b_off, W_attn, b_attn, W_out, b_out):
    bs, nq, _ = query.shape
    nv = value.shape[1]

    v = _matmul_bias(value.reshape(bs * nv, EMBED), W_value, b_value, block_m=480)
    table = v.reshape(bs * nv * HEADS, DPH)

    q2 = query.reshape(bs * nq, EMBED)
    w_cat = jnp.concatenate([W_off, W_attn], axis=1)
    b_cat = jnp.concatenate([b_off, b_attn], axis=0)
    proj = _matmul_bias(q2, w_cat, b_cat, block_m=400)
    off = proj[:, : HEADS * LEVELS * POINTS * 2].reshape(
        bs, nq, HEADS, LEVELS, POINTS, 2)
    aw = proj[:, HEADS * LEVELS * POINTS * 2:].reshape(
        bs, nq, HEADS, LEVELS * POINTS)
    aw = jax.nn.softmax(aw, axis=-1).reshape(bs, nq, HEADS, LEVELS, POINTS)

    idx, wts = _build_indices_weights(reference_points, off, aw)
    sampled = _sc_gather_weighted_sum(table, idx, wts)
    sampled = sampled.reshape(bs, HEADS, nq, DPH).transpose(0, 2, 1, 3)

    out = _matmul_bias(sampled.reshape(bs * nq, EMBED), W_out, b_out, block_m=400)
    return out.reshape(bs, nq, EMBED) + query
